# Initial kernel scaffold; baseline (speedup 1.0000x reference)
import jax
import jax.numpy as jnp
from jax.experimental import pallas as pl


def kernel(table, x):
    raise NotImplementedError("write your pallas kernel here")


# speedup vs baseline: 1.4833x; 1.4833x over previous
import functools

import jax
import jax.numpy as jnp
from jax import lax
from jax.experimental import pallas as pl
from jax.experimental.pallas import tpu as pltpu
from jax.experimental.pallas import tpu_sc as plsc

_NC = 2
_NS = 16
_NW = _NC * _NS
_IW = 128


@functools.partial(jax.jit, static_argnums=(0, 1, 2))
def _sc_gather(B, D, chunk, idx2, table):
  b_per_w = B // _NW
  nch = b_per_w // chunk
  n128 = chunk // _IW
  mesh = plsc.VectorSubcoreMesh(
      core_axis_name="c", subcore_axis_name="s",
      num_cores=_NC, num_subcores=_NS)

  @functools.partial(
      pl.kernel,
      out_type=jax.ShapeDtypeStruct((B, D), jnp.float32),
      mesh=mesh,
      compiler_params=pltpu.CompilerParams(use_tc_tiling_on_sc=False),
      scratch_types=[
          pltpu.VMEM((2, n128, _IW), jnp.int32),
          pltpu.VMEM((2, chunk, D), jnp.float32),
          pltpu.SemaphoreType.DMA,
          pltpu.SemaphoreType.DMA,
          pltpu.SemaphoreType.DMA,
          pltpu.SemaphoreType.DMA,
      ],
  )
  def body(idx_hbm, table_hbm, out_hbm, idx_v, rows_v, g0, g1, o0, o1):
    wid = lax.axis_index("s") * _NC + lax.axis_index("c")
    base = wid * b_per_w
    rbase = wid * (b_per_w // _IW)
    gsem = (g0, g1)
    osem = (o0, o1)

    def load_idx(c, sl):
      pltpu.sync_copy(idx_hbm.at[pl.ds(rbase + c * n128, n128)],
                      idx_v.at[sl])

    def start_gathers(sl):
      return [
          pltpu.async_copy(table_hbm.at[idx_v.at[sl, j]],
                           rows_v.at[sl, pl.ds(j * _IW, _IW)],
                           gsem[sl])
          for j in range(n128)
      ]

    def start_out(c, sl):
      return pltpu.async_copy(rows_v.at[sl],
                              out_hbm.at[pl.ds(base + c * chunk, chunk)],
                              osem[sl])

    gh = [None, None]
    oh = [None, None]
    load_idx(0, 0)
    gh[0] = start_gathers(0)
    for c in range(nch):
      sl = c & 1
      nsl = sl ^ 1
      if c + 1 < nch:
        if oh[nsl] is not None:
          oh[nsl].wait()
        load_idx(c + 1, nsl)
        gh[nsl] = start_gathers(nsl)
      for h in gh[sl]:
        h.wait()
      oh[sl] = start_out(c, sl)
    for h in oh:
      if h is not None:
        h.wait()

  return body(idx2, table)


def kernel(table, x):
  b, h = x.shape
  B = b * h
  D = table.shape[1]
  idx2 = x.reshape(B // _IW, _IW).astype(jnp.int32)
  out = _sc_gather(B, D, 1024, idx2, table)
  return out.reshape(b, h, D)
